# Initial kernel scaffold; baseline (speedup 1.0000x reference)
import jax
import jax.numpy as jnp
from jax.experimental import pallas as pl


def kernel(x, emb_weight):
    raise NotImplementedError("write your pallas kernel here")


# speedup vs baseline: 2.3195x; 2.3195x over previous
import jax
import jax.numpy as jnp
from jax.experimental import pallas as pl


def _copy_body(w_ref, o_ref):
    o_ref[...] = w_ref[...]


def kernel(x, emb_weight):
    seq_len = x.shape[1]
    n_embd = emb_weight.shape[1]
    block_rows = 256
    return pl.pallas_call(
        _copy_body,
        grid=(seq_len // block_rows,),
        in_specs=[pl.BlockSpec((block_rows, n_embd), lambda i: (i, 0))],
        out_specs=pl.BlockSpec((block_rows, n_embd), lambda i: (i, 0)),
        out_shape=jax.ShapeDtypeStruct((seq_len, n_embd), emb_weight.dtype),
    )(emb_weight)
